# baseline (speedup 1.0000x reference)
import functools

import jax
import jax.numpy as jnp
from jax import lax
from jax.experimental import pallas as pl
from jax.experimental.pallas import tpu as pltpu
from jax.experimental.pallas import tpu_sc as plsc

N_TOKENS = 16384
EMBED_DIM = 64
MAX_ROWS = 100000
_LANES = 16
_DB = EMBED_DIM // _LANES
_IDX_CHUNK = 128
_TR_BLOCK = 2048


def _hsum(x):
    lanes = lax.iota(jnp.int32, _LANES)
    for k in (8, 4, 2, 1):
        perm = lax.bitwise_xor(lanes, jnp.int32(k))
        x = x + x.at[perm].get(mode="promise_in_bounds")
    return x


def _rsqrt_newton(x):
    i = lax.bitcast_convert_type(x, jnp.int32)
    i = jnp.int32(0x5F3759DF) - lax.shift_right_arithmetic(i, 1)
    y = lax.bitcast_convert_type(i, jnp.float32)
    for _ in range(2):
        y = y * (1.5 - 0.5 * x * y * y)
    return y


def _tr_body(xt_ref, eye_ref, out_ref):
    out_ref[...] = lax.dot_general(
        xt_ref[...], eye_ref[...],
        dimension_numbers=(((0,), (0,)), ((), ())),
        preferred_element_type=jnp.float32)


@functools.lru_cache(maxsize=None)
def _build_tc_prep():
    grid = pl.cdiv(MAX_ROWS, _TR_BLOCK)
    return pl.pallas_call(
        _tr_body,
        grid=(grid,),
        in_specs=[
            pl.BlockSpec((EMBED_DIM, _TR_BLOCK), lambda i: (0, i)),
            pl.BlockSpec((EMBED_DIM, EMBED_DIM), lambda i: (0, 0)),
        ],
        out_specs=pl.BlockSpec((_TR_BLOCK, EMBED_DIM), lambda i: (i, 0)),
        out_shape=jax.ShapeDtypeStruct((MAX_ROWS, EMBED_DIM), jnp.float32),
    )


@functools.lru_cache(maxsize=None)
def _build_sc_kernel():
    info = plsc.get_sparse_core_info()
    nc, ns = info.num_cores, info.num_subcores
    nw = nc * ns
    bpw = N_TOKENS // nw
    n_chunks = bpw // _IDX_CHUNK
    mesh = plsc.VectorSubcoreMesh(core_axis_name="c", subcore_axis_name="s")

    @functools.partial(
        pl.kernel,
        mesh=mesh,
        compiler_params=pltpu.CompilerParams(use_tc_tiling_on_sc=False),
        out_type=jax.ShapeDtypeStruct((N_TOKENS, EMBED_DIM), jnp.float32),
        scratch_types=[
            pltpu.VMEM((n_chunks, _IDX_CHUNK), jnp.int32),
            pltpu.VMEM((n_chunks, _IDX_CHUNK), jnp.int32),
            pltpu.VMEM((n_chunks, _IDX_CHUNK), jnp.int32),
            pltpu.VMEM((bpw // _LANES, _LANES), jnp.int32),
            pltpu.VMEM((bpw // _LANES, _LANES), jnp.float32),
            pltpu.VMEM((bpw, EMBED_DIM), jnp.float32),
            pltpu.VMEM((bpw, EMBED_DIM), jnp.float32),
            pltpu.VMEM((bpw, EMBED_DIM), jnp.float32),
            pltpu.VMEM((5, EMBED_DIM), jnp.float32),
            pltpu.VMEM((8, EMBED_DIM), jnp.float32),
            pltpu.SemaphoreType.DMA,
        ],
    )
    def sc_kernel(types_hbm, tvals_hbm, nidx_hbm, i1_hbm, i2_hbm,
                  te_hbm, par_hbm, ntab_hbm, t1_hbm, t2_hbm, out_hbm,
                  nidx_v, i1_v, i2_v, types_v, tvals_v,
                  rows_n, rows_1, rows_2, te_v, par_v, sem):
        wid = lax.axis_index("s") * nc + lax.axis_index("c")
        base = wid * bpw
        cbase = wid * n_chunks

        pltpu.sync_copy(nidx_hbm.at[pl.ds(cbase, n_chunks)], nidx_v)
        pltpu.sync_copy(i1_hbm.at[pl.ds(cbase, n_chunks)], i1_v)
        pltpu.sync_copy(i2_hbm.at[pl.ds(cbase, n_chunks)], i2_v)
        gbase = wid * (bpw // _LANES)
        pltpu.sync_copy(types_hbm.at[pl.ds(gbase, bpw // _LANES)], types_v)
        pltpu.sync_copy(tvals_hbm.at[pl.ds(gbase, bpw // _LANES)], tvals_v)
        pltpu.sync_copy(te_hbm, te_v)
        pltpu.sync_copy(par_hbm, par_v)

        handles = []
        for j in range(n_chunks):
            dst = pl.ds(j * _IDX_CHUNK, _IDX_CHUNK)
            handles.append(pltpu.async_copy(ntab_hbm.at[nidx_v.at[j]], rows_n.at[dst], sem))
            handles.append(pltpu.async_copy(t1_hbm.at[i1_v.at[j]], rows_1.at[dst], sem))
            handles.append(pltpu.async_copy(t2_hbm.at[i2_v.at[j]], rows_2.at[dst], sem))
        for h in handles:
            h.wait()

        vW2 = [par_v[0, pl.ds(db * _LANES, _LANES)] for db in range(_DB)]
        vb2 = [par_v[1, pl.ds(db * _LANES, _LANES)] for db in range(_DB)]
        gam = [par_v[2, pl.ds(db * _LANES, _LANES)] for db in range(_DB)]
        bet = [par_v[3, pl.ds(db * _LANES, _LANES)] for db in range(_DB)]

        def body(g, carry):
            ty16 = types_v[g]
            tv16 = tvals_v[g]
            for l in range(_LANES):
                t = g * _LANES + l
                tvb = jnp.full((_LANES,), tv16[l])
                ty_s = ty16[l]
                accs = []
                for db in range(_DB):
                    sl = pl.ds(db * _LANES, _LANES)
                    acc = (te_v[ty_s, sl] + tvb * vW2[db] + vb2[db]
                           + rows_n[t, sl] + rows_1[t, sl] + rows_2[t, sl])
                    accs.append(acc)
                s = (accs[0] + accs[1]) + (accs[2] + accs[3])
                mu = _hsum(s) * (1.0 / EMBED_DIM)
                d = [accs[db] - mu for db in range(_DB)]
                sq = (d[0] * d[0] + d[1] * d[1]) + (d[2] * d[2] + d[3] * d[3])
                var = _hsum(sq) * (1.0 / EMBED_DIM)
                inv = _rsqrt_newton(var + 1e-5)
                for db in range(_DB):
                    rows_n[t, pl.ds(db * _LANES, _LANES)] = d[db] * inv * gam[db] + bet[db]
            return carry

        lax.fori_loop(0, bpw // _LANES, body, jnp.int32(0))
        pltpu.sync_copy(rows_n, out_hbm.at[pl.ds(base, bpw)])

    return sc_kernel


def kernel(token_types, token_values, node_indices, input1_indices, input2_indices,
           token_emb, value_W, value_b, node_idx_emb, input1_emb, input2_emb,
           combination_weights, ln_gamma, ln_beta):
    sc_kernel = _build_sc_kernel()
    tc_prep = _build_tc_prep()
    cw = combination_weights
    te_w = token_emb * cw[0][None, :]
    vW2 = value_W[:, 0] * cw[1]
    vb2 = value_b * cw[1]
    params = jnp.concatenate([
        jnp.stack([vW2, vb2, ln_gamma, ln_beta]),
        jnp.zeros((4, EMBED_DIM), jnp.float32)], axis=0)
    ntab = tc_prep(jnp.swapaxes(node_idx_emb, 0, 1), jnp.diag(cw[2]))
    t1 = tc_prep(jnp.swapaxes(input1_emb, 0, 1), jnp.diag(cw[3]))
    t2 = tc_prep(jnp.swapaxes(input2_emb, 0, 1), jnp.diag(cw[4]))
    tvals = token_values[:, 0].reshape(-1, _LANES)
    nidx = node_indices.astype(jnp.int32).reshape(-1, _IDX_CHUNK)
    i1 = input1_indices.astype(jnp.int32).reshape(-1, _IDX_CHUNK)
    i2 = input2_indices.astype(jnp.int32).reshape(-1, _IDX_CHUNK)
    ttypes = token_types.astype(jnp.int32).reshape(-1, _LANES)
    return sc_kernel(ttypes, tvals, nidx, i1, i2, te_w, params, ntab, t1, t2)


# speedup vs baseline: 1.7040x; 1.7040x over previous
import functools

import jax
import jax.numpy as jnp
from jax import lax
from jax.experimental import pallas as pl
from jax.experimental.pallas import tpu as pltpu
from jax.experimental.pallas import tpu_sc as plsc

N_TOKENS = 16384
EMBED_DIM = 64
MAX_ROWS = 100000
HALF_ROWS = 50176
_LANES = 16
_DB = EMBED_DIM // _LANES
_IDX_CHUNK = 128
_HALF = 256
_TR_BLOCK = 512


def _hsum(x):
    lanes = lax.iota(jnp.int32, _LANES)
    for k in (8, 4, 2, 1):
        perm = lax.bitwise_xor(lanes, jnp.int32(k))
        x = x + x.at[perm].get(mode="promise_in_bounds")
    return x


def _rsqrt_newton(x):
    i = lax.bitcast_convert_type(x, jnp.int32)
    i = jnp.int32(0x5F3759DF) - lax.shift_right_arithmetic(i, 1)
    y = lax.bitcast_convert_type(i, jnp.float32)
    for _ in range(2):
        y = y * (1.5 - 0.5 * x * y * y)
    return y


def _prep_body(xa0, xb0, e0, xa1, xb1, e1, xa2, xb2, e2, o0, o1, o2):
    dn = (((0,), (0,)), ((), ()))
    for xa, xb, e, o in ((xa0, xb0, e0, o0), (xa1, xb1, e1, o1),
                         (xa2, xb2, e2, o2)):
        ya = lax.dot_general(xa[...], e[...], dn,
                             preferred_element_type=jnp.float32)
        yb = lax.dot_general(xb[...], e[...], dn,
                             preferred_element_type=jnp.float32)
        o[:, 0:EMBED_DIM] = ya
        o[:, EMBED_DIM:2 * EMBED_DIM] = yb


@functools.lru_cache(maxsize=None)
def _build_tc_prep():
    grid = HALF_ROWS // _TR_BLOCK
    a_spec = pl.BlockSpec((EMBED_DIM, _TR_BLOCK), lambda i: (0, i))
    b_spec = pl.BlockSpec((EMBED_DIM, _TR_BLOCK),
                          lambda i: (0, i + HALF_ROWS // _TR_BLOCK))
    e_spec = pl.BlockSpec((EMBED_DIM, EMBED_DIM), lambda i: (0, 0))
    o_spec = pl.BlockSpec((_TR_BLOCK, 2 * EMBED_DIM), lambda i: (i, 0))
    o_type = jax.ShapeDtypeStruct((HALF_ROWS, 2 * EMBED_DIM), jnp.float32)
    return pl.pallas_call(
        _prep_body,
        grid=(grid,),
        in_specs=[a_spec, b_spec, e_spec] * 3,
        out_specs=(o_spec, o_spec, o_spec),
        out_shape=(o_type, o_type, o_type),
    )


@functools.lru_cache(maxsize=None)
def _build_sc_kernel():
    info = plsc.get_sparse_core_info()
    nc, ns = info.num_cores, info.num_subcores
    nw = nc * ns
    bpw = N_TOKENS // nw
    n_chunks = bpw // _IDX_CHUNK
    n_pass = bpw // _HALF
    cpp = _HALF // _IDX_CHUNK
    gpp = _HALF // _LANES
    mesh = plsc.VectorSubcoreMesh(core_axis_name="c", subcore_axis_name="s")

    @functools.partial(
        pl.kernel,
        mesh=mesh,
        compiler_params=pltpu.CompilerParams(use_tc_tiling_on_sc=False),
        out_type=jax.ShapeDtypeStruct((N_TOKENS // 2, 2 * EMBED_DIM),
                                      jnp.float32),
        scratch_types=[
            pltpu.VMEM((n_chunks, _IDX_CHUNK), jnp.int32),
            pltpu.VMEM((n_chunks, _IDX_CHUNK), jnp.int32),
            pltpu.VMEM((n_chunks, _IDX_CHUNK), jnp.int32),
            pltpu.VMEM((bpw // _LANES, _LANES), jnp.int32),
            pltpu.VMEM((bpw // _LANES, _LANES), jnp.int32),
            pltpu.VMEM((bpw // _LANES, _LANES), jnp.int32),
            pltpu.VMEM((bpw // _LANES, _LANES), jnp.int32),
            pltpu.VMEM((bpw // _LANES, _LANES), jnp.float32),
            pltpu.VMEM((_HALF, 2 * EMBED_DIM), jnp.float32),
            pltpu.VMEM((_HALF, 2 * EMBED_DIM), jnp.float32),
            pltpu.VMEM((_HALF, 2 * EMBED_DIM), jnp.float32),
            pltpu.VMEM((_HALF // 2, 2 * EMBED_DIM), jnp.float32),
            pltpu.VMEM((5, EMBED_DIM), jnp.float32),
            pltpu.VMEM((8, EMBED_DIM), jnp.float32),
            pltpu.SemaphoreType.DMA,
        ],
    )
    def sc_kernel(types_hbm, tvals_hbm, nidx_hbm, i1_hbm, i2_hbm,
                  noff_hbm, o1_hbm, o2_hbm, te_hbm, par_hbm,
                  ntab_hbm, t1_hbm, t2_hbm, out_hbm,
                  nidx_v, i1_v, i2_v, noffv, o1v, o2v, types_v, tvals_v,
                  rows_n, rows_1, rows_2, out_v, te_v, par_v, sem):
        wid = lax.axis_index("s") * nc + lax.axis_index("c")
        cbase = wid * n_chunks
        gbase = wid * (bpw // _LANES)

        pltpu.sync_copy(nidx_hbm.at[pl.ds(cbase, n_chunks)], nidx_v)
        pltpu.sync_copy(i1_hbm.at[pl.ds(cbase, n_chunks)], i1_v)
        pltpu.sync_copy(i2_hbm.at[pl.ds(cbase, n_chunks)], i2_v)
        pltpu.sync_copy(noff_hbm.at[pl.ds(gbase, bpw // _LANES)], noffv)
        pltpu.sync_copy(o1_hbm.at[pl.ds(gbase, bpw // _LANES)], o1v)
        pltpu.sync_copy(o2_hbm.at[pl.ds(gbase, bpw // _LANES)], o2v)
        pltpu.sync_copy(types_hbm.at[pl.ds(gbase, bpw // _LANES)], types_v)
        pltpu.sync_copy(tvals_hbm.at[pl.ds(gbase, bpw // _LANES)], tvals_v)
        pltpu.sync_copy(te_hbm, te_v)
        pltpu.sync_copy(par_hbm, par_v)

        vW2 = [par_v[0, pl.ds(db * _LANES, _LANES)] for db in range(_DB)]
        vb2 = [par_v[1, pl.ds(db * _LANES, _LANES)] for db in range(_DB)]
        gam = [par_v[2, pl.ds(db * _LANES, _LANES)] for db in range(_DB)]
        bet = [par_v[3, pl.ds(db * _LANES, _LANES)] for db in range(_DB)]

        for p in range(n_pass):
            handles = []
            for jj in range(cpp):
                j = p * cpp + jj
                dst = pl.ds(jj * _IDX_CHUNK, _IDX_CHUNK)
                handles.append(pltpu.async_copy(ntab_hbm.at[nidx_v.at[j]], rows_n.at[dst], sem))
                handles.append(pltpu.async_copy(t1_hbm.at[i1_v.at[j]], rows_1.at[dst], sem))
                handles.append(pltpu.async_copy(t2_hbm.at[i2_v.at[j]], rows_2.at[dst], sem))
            for h in handles:
                h.wait()

            def body(g, carry):
                gg = p * gpp + g
                ty16 = types_v[gg]
                tv16 = tvals_v[gg]
                on16 = noffv[gg]
                o116 = o1v[gg]
                o216 = o2v[gg]
                for l in range(_LANES):
                    t = g * _LANES + l
                    tvb = jnp.full((_LANES,), tv16[l])
                    ty_s = ty16[l]
                    on = on16[l]
                    o1 = o116[l]
                    o2 = o216[l]
                    accs = []
                    for db in range(_DB):
                        sl = pl.ds(db * _LANES, _LANES)
                        acc = (te_v[ty_s, sl] + tvb * vW2[db] + vb2[db]
                               + rows_n[t, pl.ds(on + db * _LANES, _LANES)]
                               + rows_1[t, pl.ds(o1 + db * _LANES, _LANES)]
                               + rows_2[t, pl.ds(o2 + db * _LANES, _LANES)])
                        accs.append(acc)
                    s = (accs[0] + accs[1]) + (accs[2] + accs[3])
                    mu = _hsum(s) * (1.0 / EMBED_DIM)
                    d = [accs[db] - mu for db in range(_DB)]
                    sq = (d[0] * d[0] + d[1] * d[1]) + (d[2] * d[2] + d[3] * d[3])
                    var = _hsum(sq) * (1.0 / EMBED_DIM)
                    inv = _rsqrt_newton(var + 1e-5)
                    orow = g * (_LANES // 2) + l // 2
                    for db in range(_DB):
                        col = (l % 2) * EMBED_DIM + db * _LANES
                        out_v[orow, pl.ds(col, _LANES)] = (
                            d[db] * inv * gam[db] + bet[db])
                return carry

            lax.fori_loop(0, gpp, body, jnp.int32(0))
            prow = wid * (bpw // 2) + p * (_HALF // 2)
            pltpu.sync_copy(out_v, out_hbm.at[pl.ds(prow, _HALF // 2)])

    return sc_kernel


def kernel(token_types, token_values, node_indices, input1_indices, input2_indices,
           token_emb, value_W, value_b, node_idx_emb, input1_emb, input2_emb,
           combination_weights, ln_gamma, ln_beta):
    sc_kernel = _build_sc_kernel()
    tc_prep = _build_tc_prep()
    cw = combination_weights
    te_w = token_emb * cw[0][None, :]
    vW2 = value_W[:, 0] * cw[1]
    vb2 = value_b * cw[1]
    params = jnp.concatenate([
        jnp.stack([vW2, vb2, ln_gamma, ln_beta]),
        jnp.zeros((4, EMBED_DIM), jnp.float32)], axis=0)
    nT = jnp.swapaxes(node_idx_emb, 0, 1)
    t1T = jnp.swapaxes(input1_emb, 0, 1)
    t2T = jnp.swapaxes(input2_emb, 0, 1)
    ntab, t1, t2 = tc_prep(nT, nT, jnp.diag(cw[2]),
                           t1T, t1T, jnp.diag(cw[3]),
                           t2T, t2T, jnp.diag(cw[4]))
    ni = node_indices.astype(jnp.int32)
    x1 = input1_indices.astype(jnp.int32)
    x2 = input2_indices.astype(jnp.int32)
    half = jnp.int32(HALF_ROWS)
    nidx = jnp.where(ni >= half, ni - half, ni).reshape(-1, _IDX_CHUNK)
    i1 = jnp.where(x1 >= half, x1 - half, x1).reshape(-1, _IDX_CHUNK)
    i2 = jnp.where(x2 >= half, x2 - half, x2).reshape(-1, _IDX_CHUNK)
    noff = jnp.where(ni >= half, 64, 0).astype(jnp.int32).reshape(-1, _LANES)
    o1 = jnp.where(x1 >= half, 64, 0).astype(jnp.int32).reshape(-1, _LANES)
    o2 = jnp.where(x2 >= half, 64, 0).astype(jnp.int32).reshape(-1, _LANES)
    ttypes = token_types.astype(jnp.int32).reshape(-1, _LANES)
    tvals = token_values[:, 0].reshape(-1, _LANES)
    out = sc_kernel(ttypes, tvals, nidx, i1, i2, noff, o1, o2, te_w, params,
                    ntab, t1, t2)
    return out.reshape(N_TOKENS, EMBED_DIM)


# speedup vs baseline: 2.1852x; 1.2824x over previous
import functools

import jax
import jax.numpy as jnp
from jax import lax
from jax.experimental import pallas as pl
from jax.experimental.pallas import tpu as pltpu
from jax.experimental.pallas import tpu_sc as plsc

N_TOKENS = 16384
EMBED_DIM = 64
MAX_ROWS = 100000
HALF_ROWS = 50176
_LANES = 16
_DB = EMBED_DIM // _LANES
_IDX_CHUNK = 128
_HALF = 256
_TR_BLOCK = 3584


def _hsum(x):
    lanes = lax.iota(jnp.int32, _LANES)
    for k in (8, 4, 2, 1):
        perm = lax.bitwise_xor(lanes, jnp.int32(k))
        x = x + x.at[perm].get(mode="promise_in_bounds")
    return x


def _rsqrt_newton(x):
    i = lax.bitcast_convert_type(x, jnp.int32)
    i = jnp.int32(0x5F3759DF) - lax.shift_right_arithmetic(i, 1)
    y = lax.bitcast_convert_type(i, jnp.float32)
    for _ in range(2):
        y = y * (1.5 - 0.5 * x * y * y)
    return y


def _prep_body(xa0, xb0, e0, xa1, xb1, e1, xa2, xb2, e2, o0, o1, o2):
    dn = (((0,), (0,)), ((), ()))
    for xa, xb, e, o in ((xa0, xb0, e0, o0), (xa1, xb1, e1, o1),
                         (xa2, xb2, e2, o2)):
        ya = lax.dot_general(xa[...], e[...], dn,
                             preferred_element_type=jnp.float32)
        yb = lax.dot_general(xb[...], e[...], dn,
                             preferred_element_type=jnp.float32)
        o[:, 0:EMBED_DIM] = ya
        o[:, EMBED_DIM:2 * EMBED_DIM] = yb


@functools.lru_cache(maxsize=None)
def _build_tc_prep():
    grid = HALF_ROWS // _TR_BLOCK
    a_spec = pl.BlockSpec((EMBED_DIM, _TR_BLOCK), lambda i: (0, i))
    b_spec = pl.BlockSpec((EMBED_DIM, _TR_BLOCK),
                          lambda i: (0, i + HALF_ROWS // _TR_BLOCK))
    e_spec = pl.BlockSpec((EMBED_DIM, EMBED_DIM), lambda i: (0, 0))
    o_spec = pl.BlockSpec((_TR_BLOCK, 2 * EMBED_DIM), lambda i: (i, 0))
    o_type = jax.ShapeDtypeStruct((HALF_ROWS, 2 * EMBED_DIM), jnp.float32)
    return pl.pallas_call(
        _prep_body,
        grid=(grid,),
        in_specs=[a_spec, b_spec, e_spec] * 3,
        out_specs=(o_spec, o_spec, o_spec),
        out_shape=(o_type, o_type, o_type),
    )


@functools.lru_cache(maxsize=None)
def _build_sc_kernel():
    info = plsc.get_sparse_core_info()
    nc, ns = info.num_cores, info.num_subcores
    nw = nc * ns
    bpw = N_TOKENS // nw
    n_chunks = bpw // _IDX_CHUNK
    n_pass = bpw // _HALF
    cpp = _HALF // _IDX_CHUNK
    gpp = _HALF // _LANES
    mesh = plsc.VectorSubcoreMesh(core_axis_name="c", subcore_axis_name="s")

    @functools.partial(
        pl.kernel,
        mesh=mesh,
        compiler_params=pltpu.CompilerParams(use_tc_tiling_on_sc=False),
        out_type=jax.ShapeDtypeStruct((N_TOKENS // 2, 2 * EMBED_DIM),
                                      jnp.float32),
        scratch_types=[
            pltpu.VMEM((n_chunks, _IDX_CHUNK), jnp.int32),
            pltpu.VMEM((n_chunks, _IDX_CHUNK), jnp.int32),
            pltpu.VMEM((n_chunks, _IDX_CHUNK), jnp.int32),
            pltpu.VMEM((bpw // _LANES, _LANES), jnp.int32),
            pltpu.VMEM((bpw // _LANES, _LANES), jnp.int32),
            pltpu.VMEM((bpw // _LANES, _LANES), jnp.int32),
            pltpu.VMEM((bpw // _LANES, _LANES), jnp.int32),
            pltpu.VMEM((bpw // _LANES, _LANES), jnp.float32),
            pltpu.VMEM((_HALF, 2 * EMBED_DIM), jnp.float32),
            pltpu.VMEM((_HALF, 2 * EMBED_DIM), jnp.float32),
            pltpu.VMEM((_HALF, 2 * EMBED_DIM), jnp.float32),
            pltpu.VMEM((_HALF // 2, 2 * EMBED_DIM), jnp.float32),
            pltpu.VMEM((5, EMBED_DIM), jnp.float32),
            pltpu.VMEM((8, EMBED_DIM), jnp.float32),
            pltpu.SemaphoreType.DMA,
        ],
    )
    def sc_kernel(types_hbm, tvals_hbm, nidx_hbm, i1_hbm, i2_hbm,
                  noff_hbm, o1_hbm, o2_hbm, te_hbm, par_hbm,
                  ntab_hbm, t1_hbm, t2_hbm, out_hbm,
                  nidx_v, i1_v, i2_v, noffv, o1v, o2v, types_v, tvals_v,
                  rows_n, rows_1, rows_2, out_v, te_v, par_v, sem):
        wid = lax.axis_index("s") * nc + lax.axis_index("c")
        cbase = wid * n_chunks
        gbase = wid * (bpw // _LANES)

        pltpu.sync_copy(nidx_hbm.at[pl.ds(cbase, n_chunks)], nidx_v)
        pltpu.sync_copy(i1_hbm.at[pl.ds(cbase, n_chunks)], i1_v)
        pltpu.sync_copy(i2_hbm.at[pl.ds(cbase, n_chunks)], i2_v)
        pltpu.sync_copy(noff_hbm.at[pl.ds(gbase, bpw // _LANES)], noffv)
        pltpu.sync_copy(o1_hbm.at[pl.ds(gbase, bpw // _LANES)], o1v)
        pltpu.sync_copy(o2_hbm.at[pl.ds(gbase, bpw // _LANES)], o2v)
        pltpu.sync_copy(types_hbm.at[pl.ds(gbase, bpw // _LANES)], types_v)
        pltpu.sync_copy(tvals_hbm.at[pl.ds(gbase, bpw // _LANES)], tvals_v)
        pltpu.sync_copy(te_hbm, te_v)
        pltpu.sync_copy(par_hbm, par_v)

        vW2 = [par_v[0, pl.ds(db * _LANES, _LANES)] for db in range(_DB)]
        vb2 = [par_v[1, pl.ds(db * _LANES, _LANES)] for db in range(_DB)]
        gam = [par_v[2, pl.ds(db * _LANES, _LANES)] for db in range(_DB)]
        bet = [par_v[3, pl.ds(db * _LANES, _LANES)] for db in range(_DB)]

        for p in range(n_pass):
            handles = []
            for jj in range(cpp):
                j = p * cpp + jj
                dst = pl.ds(jj * _IDX_CHUNK, _IDX_CHUNK)
                handles.append(pltpu.async_copy(ntab_hbm.at[nidx_v.at[j]], rows_n.at[dst], sem))
                handles.append(pltpu.async_copy(t1_hbm.at[i1_v.at[j]], rows_1.at[dst], sem))
                handles.append(pltpu.async_copy(t2_hbm.at[i2_v.at[j]], rows_2.at[dst], sem))
            for h in handles:
                h.wait()

            def body(g, carry):
                gg = p * gpp + g
                ty16 = types_v[gg]
                tv16 = tvals_v[gg]
                on16 = noffv[gg]
                o116 = o1v[gg]
                o216 = o2v[gg]
                for l in range(_LANES):
                    t = g * _LANES + l
                    tvb = jnp.full((_LANES,), tv16[l])
                    ty_s = ty16[l]
                    on = on16[l]
                    o1 = o116[l]
                    o2 = o216[l]
                    accs = []
                    for db in range(_DB):
                        sl = pl.ds(db * _LANES, _LANES)
                        acc = (te_v[ty_s, sl] + tvb * vW2[db] + vb2[db]
                               + rows_n[t, pl.ds(on + db * _LANES, _LANES)]
                               + rows_1[t, pl.ds(o1 + db * _LANES, _LANES)]
                               + rows_2[t, pl.ds(o2 + db * _LANES, _LANES)])
                        accs.append(acc)
                    s = (accs[0] + accs[1]) + (accs[2] + accs[3])
                    mu = _hsum(s) * (1.0 / EMBED_DIM)
                    d = [accs[db] - mu for db in range(_DB)]
                    sq = (d[0] * d[0] + d[1] * d[1]) + (d[2] * d[2] + d[3] * d[3])
                    var = _hsum(sq) * (1.0 / EMBED_DIM)
                    inv = _rsqrt_newton(var + 1e-5)
                    orow = g * (_LANES // 2) + l // 2
                    for db in range(_DB):
                        col = (l % 2) * EMBED_DIM + db * _LANES
                        out_v[orow, pl.ds(col, _LANES)] = (
                            d[db] * inv * gam[db] + bet[db])
                return carry

            lax.fori_loop(0, gpp, body, jnp.int32(0))
            prow = wid * (bpw // 2) + p * (_HALF // 2)
            pltpu.sync_copy(out_v, out_hbm.at[pl.ds(prow, _HALF // 2)])

    return sc_kernel


def kernel(token_types, token_values, node_indices, input1_indices, input2_indices,
           token_emb, value_W, value_b, node_idx_emb, input1_emb, input2_emb,
           combination_weights, ln_gamma, ln_beta):
    sc_kernel = _build_sc_kernel()
    tc_prep = _build_tc_prep()
    cw = combination_weights
    te_w = token_emb * cw[0][None, :]
    vW2 = value_W[:, 0] * cw[1]
    vb2 = value_b * cw[1]
    params = jnp.concatenate([
        jnp.stack([vW2, vb2, ln_gamma, ln_beta]),
        jnp.zeros((4, EMBED_DIM), jnp.float32)], axis=0)
    nT = jnp.swapaxes(node_idx_emb, 0, 1)
    t1T = jnp.swapaxes(input1_emb, 0, 1)
    t2T = jnp.swapaxes(input2_emb, 0, 1)
    ntab, t1, t2 = tc_prep(nT, nT, jnp.diag(cw[2]),
                           t1T, t1T, jnp.diag(cw[3]),
                           t2T, t2T, jnp.diag(cw[4]))
    ni = node_indices.astype(jnp.int32)
    x1 = input1_indices.astype(jnp.int32)
    x2 = input2_indices.astype(jnp.int32)
    half = jnp.int32(HALF_ROWS)
    nidx = jnp.where(ni >= half, ni - half, ni).reshape(-1, _IDX_CHUNK)
    i1 = jnp.where(x1 >= half, x1 - half, x1).reshape(-1, _IDX_CHUNK)
    i2 = jnp.where(x2 >= half, x2 - half, x2).reshape(-1, _IDX_CHUNK)
    noff = jnp.where(ni >= half, 64, 0).astype(jnp.int32).reshape(-1, _LANES)
    o1 = jnp.where(x1 >= half, 64, 0).astype(jnp.int32).reshape(-1, _LANES)
    o2 = jnp.where(x2 >= half, 64, 0).astype(jnp.int32).reshape(-1, _LANES)
    ttypes = token_types.astype(jnp.int32).reshape(-1, _LANES)
    tvals = token_values[:, 0].reshape(-1, _LANES)
    out = sc_kernel(ttypes, tvals, nidx, i1, i2, noff, o1, o2, te_w, params,
                    ntab, t1, t2)
    return out.reshape(N_TOKENS, EMBED_DIM)


# speedup vs baseline: 2.2589x; 1.0337x over previous
import functools

import jax
import jax.numpy as jnp
from jax import lax
from jax.experimental import pallas as pl
from jax.experimental.pallas import tpu as pltpu
from jax.experimental.pallas import tpu_sc as plsc

N_TOKENS = 16384
EMBED_DIM = 64
MAX_ROWS = 100000
HALF_ROWS = 50176
_LANES = 16
_DB = EMBED_DIM // _LANES
_IDX_CHUNK = 128
_HALF = 128
_TR_BLOCK = 7168


def _hsum(x):
    lanes = lax.iota(jnp.int32, _LANES)
    for k in (8, 4, 2, 1):
        perm = lax.bitwise_xor(lanes, jnp.int32(k))
        x = x + x.at[perm].get(mode="promise_in_bounds")
    return x


def _rsqrt_newton(x):
    i = lax.bitcast_convert_type(x, jnp.int32)
    i = jnp.int32(0x5F3759DF) - lax.shift_right_arithmetic(i, 1)
    y = lax.bitcast_convert_type(i, jnp.float32)
    for _ in range(1):
        y = y * (1.5 - 0.5 * x * y * y)
    return y


def _prep_body(xa0, xb0, e0, xa1, xb1, e1, xa2, xb2, e2, o0, o1, o2):
    dn = (((0,), (0,)), ((), ()))
    for xa, xb, e, o in ((xa0, xb0, e0, o0), (xa1, xb1, e1, o1),
                         (xa2, xb2, e2, o2)):
        ya = lax.dot_general(xa[...], e[...], dn,
                             preferred_element_type=jnp.float32)
        yb = lax.dot_general(xb[...], e[...], dn,
                             preferred_element_type=jnp.float32)
        o[:, 0:EMBED_DIM] = ya
        o[:, EMBED_DIM:2 * EMBED_DIM] = yb


@functools.lru_cache(maxsize=None)
def _build_tc_prep():
    grid = HALF_ROWS // _TR_BLOCK
    a_spec = pl.BlockSpec((EMBED_DIM, _TR_BLOCK), lambda i: (0, i))
    b_spec = pl.BlockSpec((EMBED_DIM, _TR_BLOCK),
                          lambda i: (0, i + HALF_ROWS // _TR_BLOCK))
    e_spec = pl.BlockSpec((EMBED_DIM, EMBED_DIM), lambda i: (0, 0))
    o_spec = pl.BlockSpec((_TR_BLOCK, 2 * EMBED_DIM), lambda i: (i, 0))
    o_type = jax.ShapeDtypeStruct((HALF_ROWS, 2 * EMBED_DIM), jnp.float32)
    return pl.pallas_call(
        _prep_body,
        grid=(grid,),
        in_specs=[a_spec, b_spec, e_spec] * 3,
        out_specs=(o_spec, o_spec, o_spec),
        out_shape=(o_type, o_type, o_type),
    )


@functools.lru_cache(maxsize=None)
def _build_sc_kernel():
    info = plsc.get_sparse_core_info()
    nc, ns = info.num_cores, info.num_subcores
    nw = nc * ns
    bpw = N_TOKENS // nw
    n_chunks = bpw // _IDX_CHUNK
    n_pass = bpw // _HALF
    gpp = _HALF // _LANES
    mesh = plsc.VectorSubcoreMesh(core_axis_name="c", subcore_axis_name="s")

    @functools.partial(
        pl.kernel,
        mesh=mesh,
        compiler_params=pltpu.CompilerParams(use_tc_tiling_on_sc=False),
        out_type=jax.ShapeDtypeStruct((N_TOKENS // 2, 2 * EMBED_DIM),
                                      jnp.float32),
        scratch_types=[
            pltpu.VMEM((n_chunks, _IDX_CHUNK), jnp.int32),
            pltpu.VMEM((n_chunks, _IDX_CHUNK), jnp.int32),
            pltpu.VMEM((n_chunks, _IDX_CHUNK), jnp.int32),
            pltpu.VMEM((bpw // _LANES, _LANES), jnp.int32),
            pltpu.VMEM((bpw // _LANES, _LANES), jnp.int32),
            pltpu.VMEM((bpw // _LANES, _LANES), jnp.int32),
            pltpu.VMEM((bpw // _LANES, _LANES), jnp.int32),
            pltpu.VMEM((bpw // _LANES, _LANES), jnp.float32),
            pltpu.VMEM((2, _HALF, 2 * EMBED_DIM), jnp.float32),
            pltpu.VMEM((2, _HALF, 2 * EMBED_DIM), jnp.float32),
            pltpu.VMEM((2, _HALF, 2 * EMBED_DIM), jnp.float32),
            pltpu.VMEM((_HALF // 2, 2 * EMBED_DIM), jnp.float32),
            pltpu.VMEM((5, EMBED_DIM), jnp.float32),
            pltpu.VMEM((8, EMBED_DIM), jnp.float32),
            pltpu.SemaphoreType.DMA,
            pltpu.SemaphoreType.DMA,
        ],
    )
    def sc_kernel(types_hbm, tvals_hbm, nidx_hbm, i1_hbm, i2_hbm,
                  noff_hbm, o1_hbm, o2_hbm, te_hbm, par_hbm,
                  ntab_hbm, t1_hbm, t2_hbm, out_hbm,
                  nidx_v, i1_v, i2_v, noffv, o1v, o2v, types_v, tvals_v,
                  rows_n, rows_1, rows_2, out_v, te_v, par_v, sem0, sem1):
        wid = lax.axis_index("s") * nc + lax.axis_index("c")
        cbase = wid * n_chunks
        gbase = wid * (bpw // _LANES)

        pltpu.sync_copy(nidx_hbm.at[pl.ds(cbase, n_chunks)], nidx_v)
        pltpu.sync_copy(i1_hbm.at[pl.ds(cbase, n_chunks)], i1_v)
        pltpu.sync_copy(i2_hbm.at[pl.ds(cbase, n_chunks)], i2_v)
        pltpu.sync_copy(noff_hbm.at[pl.ds(gbase, bpw // _LANES)], noffv)
        pltpu.sync_copy(o1_hbm.at[pl.ds(gbase, bpw // _LANES)], o1v)
        pltpu.sync_copy(o2_hbm.at[pl.ds(gbase, bpw // _LANES)], o2v)
        pltpu.sync_copy(types_hbm.at[pl.ds(gbase, bpw // _LANES)], types_v)
        pltpu.sync_copy(tvals_hbm.at[pl.ds(gbase, bpw // _LANES)], tvals_v)
        pltpu.sync_copy(te_hbm, te_v)
        pltpu.sync_copy(par_hbm, par_v)

        vW2 = [par_v[0, pl.ds(db * _LANES, _LANES)] for db in range(_DB)]
        vb2 = [par_v[1, pl.ds(db * _LANES, _LANES)] for db in range(_DB)]
        gam = [par_v[2, pl.ds(db * _LANES, _LANES)] for db in range(_DB)]
        bet = [par_v[3, pl.ds(db * _LANES, _LANES)] for db in range(_DB)]

        sems = (sem0, sem1)

        def fire(p):
            b = p % 2
            return [
                pltpu.async_copy(ntab_hbm.at[nidx_v.at[p]], rows_n.at[b], sems[b]),
                pltpu.async_copy(t1_hbm.at[i1_v.at[p]], rows_1.at[b], sems[b]),
                pltpu.async_copy(t2_hbm.at[i2_v.at[p]], rows_2.at[b], sems[b]),
            ]

        pend = fire(0)
        for p in range(n_pass):
            for h in pend:
                h.wait()
            if p + 1 < n_pass:
                pend = fire(p + 1)
            b = p % 2

            def body(g, carry):
                gg = p * gpp + g
                ty16 = types_v[gg]
                tv16 = tvals_v[gg]
                on16 = noffv[gg]
                o116 = o1v[gg]
                o216 = o2v[gg]
                for l in range(_LANES):
                    t = g * _LANES + l
                    tvb = jnp.full((_LANES,), tv16[l])
                    ty_s = ty16[l]
                    on = on16[l]
                    o1 = o116[l]
                    o2 = o216[l]
                    accs = []
                    for db in range(_DB):
                        sl = pl.ds(db * _LANES, _LANES)
                        acc = (te_v[ty_s, sl] + tvb * vW2[db] + vb2[db]
                               + rows_n[b, t, pl.ds(on + db * _LANES, _LANES)]
                               + rows_1[b, t, pl.ds(o1 + db * _LANES, _LANES)]
                               + rows_2[b, t, pl.ds(o2 + db * _LANES, _LANES)])
                        accs.append(acc)
                    s = (accs[0] + accs[1]) + (accs[2] + accs[3])
                    mu = _hsum(s) * (1.0 / EMBED_DIM)
                    d = [accs[db] - mu for db in range(_DB)]
                    sq = (d[0] * d[0] + d[1] * d[1]) + (d[2] * d[2] + d[3] * d[3])
                    var = _hsum(sq) * (1.0 / EMBED_DIM)
                    inv = _rsqrt_newton(var + 1e-5)
                    orow = g * (_LANES // 2) + l // 2
                    for db in range(_DB):
                        col = (l % 2) * EMBED_DIM + db * _LANES
                        out_v[orow, pl.ds(col, _LANES)] = (
                            d[db] * inv * gam[db] + bet[db])
                return carry

            lax.fori_loop(0, gpp, body, jnp.int32(0))
            prow = wid * (bpw // 2) + p * (_HALF // 2)
            pltpu.sync_copy(out_v, out_hbm.at[pl.ds(prow, _HALF // 2)])

    return sc_kernel


def kernel(token_types, token_values, node_indices, input1_indices, input2_indices,
           token_emb, value_W, value_b, node_idx_emb, input1_emb, input2_emb,
           combination_weights, ln_gamma, ln_beta):
    sc_kernel = _build_sc_kernel()
    tc_prep = _build_tc_prep()
    cw = combination_weights
    te_w = token_emb * cw[0][None, :]
    vW2 = value_W[:, 0] * cw[1]
    vb2 = value_b * cw[1]
    params = jnp.concatenate([
        jnp.stack([vW2, vb2, ln_gamma, ln_beta]),
        jnp.zeros((4, EMBED_DIM), jnp.float32)], axis=0)
    nT = jnp.swapaxes(node_idx_emb, 0, 1)
    t1T = jnp.swapaxes(input1_emb, 0, 1)
    t2T = jnp.swapaxes(input2_emb, 0, 1)
    ntab, t1, t2 = tc_prep(nT, nT, jnp.diag(cw[2]),
                           t1T, t1T, jnp.diag(cw[3]),
                           t2T, t2T, jnp.diag(cw[4]))
    ni = node_indices.astype(jnp.int32)
    x1 = input1_indices.astype(jnp.int32)
    x2 = input2_indices.astype(jnp.int32)
    half = jnp.int32(HALF_ROWS)
    nidx = jnp.where(ni >= half, ni - half, ni).reshape(-1, _IDX_CHUNK)
    i1 = jnp.where(x1 >= half, x1 - half, x1).reshape(-1, _IDX_CHUNK)
    i2 = jnp.where(x2 >= half, x2 - half, x2).reshape(-1, _IDX_CHUNK)
    noff = jnp.where(ni >= half, 64, 0).astype(jnp.int32).reshape(-1, _LANES)
    o1 = jnp.where(x1 >= half, 64, 0).astype(jnp.int32).reshape(-1, _LANES)
    o2 = jnp.where(x2 >= half, 64, 0).astype(jnp.int32).reshape(-1, _LANES)
    ttypes = token_types.astype(jnp.int32).reshape(-1, _LANES)
    tvals = token_values[:, 0].reshape(-1, _LANES)
    out = sc_kernel(ttypes, tvals, nidx, i1, i2, noff, o1, o2, te_w, params,
                    ntab, t1, t2)
    return out.reshape(N_TOKENS, EMBED_DIM)
